# baseline (speedup 1.0000x reference)
import functools
import jax
import jax.numpy as jnp
from jax import lax
from jax.experimental import pallas as pl
from jax.experimental.pallas import tpu as pltpu
from jax.experimental.pallas import tpu_sc as plsc

_PADDING_IDX = 0
_NC = 2
_NS = 16
_NW = _NC * _NS
_L = 16
_C = 32


def _sc_body(seq_len, dim, in_hbm, w_hbm, out_hbm, ids_v, w_v, zero_v):
    bsz = 4
    rows_per_w = seq_len // _NW
    n_chunks = rows_per_w // _C
    wid = lax.axis_index("s") * _NC + lax.axis_index("c")

    for j in range(dim // _L):
        zero_v[0, pl.ds(j * _L, _L)] = jnp.zeros((_L,), jnp.float32)

    def chunk_body(chunk, _):
        base = wid * rows_per_w + chunk * _C
        pltpu.sync_copy(w_hbm.at[pl.ds(base, _C)], w_v)
        for b in range(bsz):
            pltpu.sync_copy(in_hbm.at[pl.ds(b * seq_len + base, _C)],
                            ids_v.at[pl.ds(b * _C, _C)])

        for b in range(bsz):
            pltpu.sync_copy(w_v, out_hbm.at[pl.ds(b * seq_len + base, _C)])

        def row_body(r, _):
            for b in range(bsz):
                vr = ids_v[pl.ds(b * _C + r, _L)]

                @pl.when(vr[0] == _PADDING_IDX)
                def _():
                    pltpu.sync_copy(
                        zero_v,
                        out_hbm.at[pl.ds(b * seq_len + base + r, 1)])
            return 0

        lax.fori_loop(0, _C, row_body, 0)
        return 0

    lax.fori_loop(0, n_chunks, chunk_body, 0)


def kernel(input, weights):
    bsz, seq_len = input.shape
    dim = weights.shape[1]
    mesh = plsc.VectorSubcoreMesh(
        core_axis_name="c", subcore_axis_name="s",
        num_cores=_NC, num_subcores=_NS)
    inp_flat = input.reshape(bsz * seq_len)
    body = functools.partial(_sc_body, seq_len, dim)
    out = pl.kernel(
        body,
        out_type=jax.ShapeDtypeStruct((bsz * seq_len, dim), weights.dtype),
        mesh=mesh,
        scratch_types=[
            pltpu.VMEM((bsz * _C + _L,), jnp.int32),
            pltpu.VMEM((_C, dim), jnp.float32),
            pltpu.VMEM((1, dim), jnp.float32),
        ],
    )(inp_flat, weights)
    return out.reshape(bsz, seq_len, dim)


# speedup vs baseline: 1.3064x; 1.3064x over previous
import functools
import jax
import jax.numpy as jnp
from jax import lax
from jax.experimental import pallas as pl
from jax.experimental.pallas import tpu as pltpu
from jax.experimental.pallas import tpu_sc as plsc

_PADDING_IDX = 0
_NC = 2
_NS = 16
_NW = _NC * _NS
_L = 16
_C = 32


def _sc_body(seq_len, dim, in_hbm, w_hbm, out_hbm,
             ids_v, w_v0, w_v1, zero_v, sem_r0, sem_r1, sem_w0, sem_w1):
    bsz = 4
    rows_per_w = seq_len // _NW
    n_chunks = rows_per_w // _C
    wid = lax.axis_index("s") * _NC + lax.axis_index("c")
    base0 = wid * rows_per_w

    for j in range(dim // _L):
        zero_v[0, pl.ds(j * _L, _L)] = jnp.zeros((_L,), jnp.float32)

    for b in range(bsz):
        pltpu.sync_copy(in_hbm.at[pl.ds(b * seq_len + base0, rows_per_w)],
                        ids_v.at[pl.ds(b * rows_per_w, rows_per_w)])

    bufs = (w_v0, w_v1)
    sems_r = (sem_r0, sem_r1)
    sems_w = (sem_w0, sem_w1)

    def fixup(c):
        def row_body(r, _):
            for b in range(bsz):
                vr = ids_v[pl.ds(b * rows_per_w + c * _C + r, _L)]

                @pl.when(vr[0] == _PADDING_IDX)
                def _():
                    pltpu.sync_copy(
                        zero_v,
                        out_hbm.at[pl.ds(b * seq_len + base0 + c * _C + r, 1)])
            return 0

        lax.fori_loop(0, _C, row_body, 0)

    r_handles = {}
    w_handles = {}
    r_handles[0] = pltpu.async_copy(
        w_hbm.at[pl.ds(base0, _C)], bufs[0], sems_r[0])
    for c in range(n_chunks):
        buf = bufs[c % 2]
        r_handles[c].wait()
        if c >= 1:
            for h in w_handles[c - 1]:
                h.wait()
        if c + 1 < n_chunks:
            r_handles[c + 1] = pltpu.async_copy(
                w_hbm.at[pl.ds(base0 + (c + 1) * _C, _C)],
                bufs[(c + 1) % 2], sems_r[(c + 1) % 2])
        w_handles[c] = [
            pltpu.async_copy(
                buf, out_hbm.at[pl.ds(b * seq_len + base0 + c * _C, _C)],
                sems_w[c % 2])
            for b in range(bsz)]
        if c >= 1:
            fixup(c - 1)
    for h in w_handles[n_chunks - 1]:
        h.wait()
    fixup(n_chunks - 1)


def kernel(input, weights):
    bsz, seq_len = input.shape
    dim = weights.shape[1]
    mesh = plsc.VectorSubcoreMesh(
        core_axis_name="c", subcore_axis_name="s",
        num_cores=_NC, num_subcores=_NS)
    inp_flat = input.reshape(bsz * seq_len)
    body = functools.partial(_sc_body, seq_len, dim)
    out = pl.kernel(
        body,
        out_type=jax.ShapeDtypeStruct((bsz * seq_len, dim), weights.dtype),
        mesh=mesh,
        scratch_types=[
            pltpu.VMEM((bsz * (seq_len // _NW) + _L,), jnp.int32),
            pltpu.VMEM((_C, dim), jnp.float32),
            pltpu.VMEM((_C, dim), jnp.float32),
            pltpu.VMEM((1, dim), jnp.float32),
            pltpu.SemaphoreType.DMA,
            pltpu.SemaphoreType.DMA,
            pltpu.SemaphoreType.DMA,
            pltpu.SemaphoreType.DMA,
        ],
    )(inp_flat, weights)
    return out.reshape(bsz, seq_len, dim)
